# baseline (speedup 1.0000x reference)
import jax
import jax.numpy as jnp
import numpy as np
from jax import lax
from jax.experimental import pallas as pl
from jax.experimental.pallas import tpu as pltpu
from jax.experimental.pallas import tpu_sc as plsc

PRIMES = [100003, 100019, 100043, 100057, 100069, 100103, 100109, 100129]
NUM_HEADS = 8
D_PER_HEAD = 128
D_MEM = 1024
D_HIDDEN = 2048
KERNEL = 4
B, L = 2, 2048
EPS = float(np.finfo(np.float32).eps)

_OFFSETS8 = np.concatenate([[0], np.cumsum(PRIMES)[:-1]]).astype(np.int32)

_OFFP = _OFFSETS8[np.arange(128) % 8].astype(np.int32)


_NC, _NS = 2, 16
_NW = _NC * _NS
_N_ROWS = B * L * NUM_HEADS
_PER_W = _N_ROWS // _NW
_CH = 128
_NCH = _PER_W // _CH


def _sc_gather_body(idx_hbm, offp_hbm, table_hbm, out_hbm,
                    idxall_v, offp_v, idxp0_v, idxp1_v, rows0_v, rows1_v,
                    sem_g0, sem_g1, sem_w0, sem_w1):
    wid = lax.axis_index("s") * _NC + lax.axis_index("c")
    pltpu.sync_copy(offp_hbm, offp_v)
    pltpu.sync_copy(idx_hbm.at[pl.ds(wid * _PER_W, _PER_W)], idxall_v)
    idxp_bufs = (idxp0_v, idxp1_v)
    rows_bufs = (rows0_v, rows1_v)
    g_sems = (sem_g0, sem_g1)
    wb_sems = (sem_w0, sem_w1)
    g = [None] * _NCH
    wb = [None] * _NCH

    def start_gather(c):
        for k in range(_CH // 16):
            sl = pl.ds(k * 16, 16)
            idxp_bufs[c % 2][sl] = (idxall_v[pl.ds(c * _CH + k * 16, 16)]
                                    + offp_v[sl])
        return pltpu.async_copy(table_hbm.at[idxp_bufs[c % 2]],
                                rows_bufs[c % 2], g_sems[c % 2])

    def start_wb(c):
        base = wid * _PER_W + c * _CH
        return pltpu.async_copy(rows_bufs[c % 2],
                                out_hbm.at[pl.ds(base, _CH)], wb_sems[c % 2])

    for c in range(_NCH):
        if c >= 2:
            wb[c - 2].wait()
        g[c] = start_gather(c)
        if c >= 1:
            g[c - 1].wait()
            wb[c - 1] = start_wb(c - 1)
    g[_NCH - 1].wait()
    wb[_NCH - 1] = start_wb(_NCH - 1)
    wb[_NCH - 2].wait()
    wb[_NCH - 1].wait()


def _sc_gather(flat_idx, offp, emb):
    gather = pl.kernel(
        _sc_gather_body,
        out_type=jax.ShapeDtypeStruct((_N_ROWS, D_PER_HEAD), jnp.float32),
        mesh=plsc.VectorSubcoreMesh(core_axis_name="c", subcore_axis_name="s",
                                    num_cores=_NC, num_subcores=_NS),
        scratch_types=[
            pltpu.VMEM((_PER_W,), jnp.int32),
            pltpu.VMEM((_CH,), jnp.int32),
            pltpu.VMEM((_CH,), jnp.int32),
            pltpu.VMEM((_CH,), jnp.int32),
            pltpu.VMEM((_CH, D_PER_HEAD), jnp.float32),
            pltpu.VMEM((_CH, D_PER_HEAD), jnp.float32),
            pltpu.SemaphoreType.DMA,
            pltpu.SemaphoreType.DMA,
            pltpu.SemaphoreType.DMA,
            pltpu.SemaphoreType.DMA,
        ],
    )
    return gather(flat_idx, offp, emb)


_TL = 256
_NT = L // _TL


def _rms(x, g):
    ms = jnp.mean(x * x, axis=-1, keepdims=True)
    return x * lax.rsqrt(ms + EPS) * g


def _gate_vtilde(e, h, wv, wk, ghk):
    dn = (((1,), (1,)), ((), ()))
    kt = lax.dot_general(e, wk, dn, precision=lax.Precision.DEFAULT,
                         preferred_element_type=jnp.float32)
    vt = lax.dot_general(e, wv, dn, precision=lax.Precision.DEFAULT,
                         preferred_element_type=jnp.float32)
    ms_h = jnp.mean(h * h, axis=-1, keepdims=True)
    ms_k = jnp.mean(kt * kt, axis=-1, keepdims=True)
    hk = jnp.sum(h * kt * ghk, axis=-1, keepdims=True)
    dot = hk * lax.rsqrt(ms_h + EPS) * lax.rsqrt(ms_k + EPS) * (D_HIDDEN ** -0.5)
    sdp = jnp.sqrt(jnp.clip(jnp.abs(dot), 1e-6, None)) * jnp.sign(dot)
    return jax.nn.sigmoid(sdp) * vt


def _tc_body(e_ref, ep_ref, h_ref, hp_ref, wv_ref, wk_ref, gh_ref, gk_ref,
             gv_ref, cw_ref, cb_ref, out_ref):
    i = pl.program_id(1)
    wv = wv_ref[...]
    wk = wk_ref[...]
    ghk = gh_ref[...] * gk_ref[...]
    gv = gv_ref[...]

    h = h_ref[0]
    e = jnp.reshape(e_ref[0], (_TL, D_MEM))
    v_tilde = _gate_vtilde(e, h, wv, wk, ghk)

    cw = cw_ref[...]
    cb = cb_ref[...]
    conv_active = jnp.max(jnp.abs(cw)) > 0.0

    @pl.when(jnp.logical_not(conv_active))
    def _skip_conv():
        y = cb * jax.nn.sigmoid(cb)
        out_ref[0] = h + y + v_tilde

    @pl.when(conv_active)
    def _full_conv():
        vn = _rms(v_tilde, gv)
        ep = jnp.reshape(ep_ref[0], (8, D_MEM))
        vtp = _gate_vtilde(ep, hp_ref[0], wv, wk, ghk)
        vnp = _rms(vtp, gv)
        vnp = jnp.where(i == 0, jnp.zeros_like(vnp), vnp)

        ext = jnp.concatenate([vnp, vn], axis=0)
        y = cb
        for j in range(KERNEL):
            y = y + cw[j:j + 1] * ext[5 + j:5 + j + _TL]
        y = y * jax.nn.sigmoid(y)
        out_ref[0] = h + y + v_tilde


def _tc_dense(e3, hidden, wv, wk, gh, gk, gv, cw, cb):
    grid = (B, _NT)
    return pl.pallas_call(
        _tc_body,
        grid=grid,
        in_specs=[
            pl.BlockSpec((1, _TL * NUM_HEADS, D_PER_HEAD),
                         lambda b, i: (b, i, 0)),
            pl.BlockSpec((1, 8 * NUM_HEADS, D_PER_HEAD),
                         lambda b, i: (b, jnp.maximum(i * (_TL // 8) - 1, 0), 0)),
            pl.BlockSpec((1, _TL, D_HIDDEN), lambda b, i: (b, i, 0)),
            pl.BlockSpec((1, 8, D_HIDDEN),
                         lambda b, i: (b, jnp.maximum(i * (_TL // 8) - 1, 0), 0)),
            pl.BlockSpec((D_HIDDEN, D_MEM), lambda b, i: (0, 0)),
            pl.BlockSpec((D_HIDDEN, D_MEM), lambda b, i: (0, 0)),
            pl.BlockSpec((1, D_HIDDEN), lambda b, i: (0, 0)),
            pl.BlockSpec((1, D_HIDDEN), lambda b, i: (0, 0)),
            pl.BlockSpec((1, D_HIDDEN), lambda b, i: (0, 0)),
            pl.BlockSpec((KERNEL, D_HIDDEN), lambda b, i: (0, 0)),
            pl.BlockSpec((1, D_HIDDEN), lambda b, i: (0, 0)),
        ],
        out_specs=pl.BlockSpec((1, _TL, D_HIDDEN), lambda b, i: (b, i, 0)),
        out_shape=jax.ShapeDtypeStruct((B, L, D_HIDDEN), jnp.float32),
    )(e3, e3, hidden, hidden, wv, wk, gh, gk, gv, cw, cb)


def kernel(engram_hash_indices, hidden_states, emb, w_v, w_k, g_h, g_k, g_v,
           conv_w, conv_b):
    flat_idx = engram_hash_indices.reshape(-1)
    rows = _sc_gather(flat_idx, jnp.asarray(_OFFP), emb)
    e4 = rows.reshape(B, L * NUM_HEADS, D_PER_HEAD)
    out = _tc_dense(
        e4,
        hidden_states,
        w_v,
        w_k,
        g_h.reshape(1, D_HIDDEN),
        g_k.reshape(1, D_HIDDEN),
        g_v.reshape(1, D_HIDDEN),
        jnp.transpose(conv_w[:, 0, :], (1, 0)),
        conv_b.reshape(1, D_HIDDEN),
    )
    return out


# speedup vs baseline: 1.0153x; 1.0153x over previous
import jax
import jax.numpy as jnp
import numpy as np
from jax import lax
from jax.experimental import pallas as pl
from jax.experimental.pallas import tpu as pltpu
from jax.experimental.pallas import tpu_sc as plsc

PRIMES = [100003, 100019, 100043, 100057, 100069, 100103, 100109, 100129]
NUM_HEADS = 8
D_PER_HEAD = 128
D_MEM = 1024
D_HIDDEN = 2048
KERNEL = 4
B, L = 2, 2048
EPS = float(np.finfo(np.float32).eps)

_OFFSETS8 = np.concatenate([[0], np.cumsum(PRIMES)[:-1]]).astype(np.int32)

_OFFP = _OFFSETS8[np.arange(128) % 8].astype(np.int32)


_NC, _NS = 2, 16
_NW = _NC * _NS
_N_ROWS = B * L * NUM_HEADS
_PER_W = _N_ROWS // _NW
_CH = 128
_NCH = _PER_W // _CH


def _sc_gather_body(idx_hbm, offp_hbm, table_hbm, out_hbm,
                    idxall_v, offp_v, idxp0_v, idxp1_v, rows0_v, rows1_v,
                    sem_g0, sem_g1, sem_w0, sem_w1):
    wid = lax.axis_index("s") * _NC + lax.axis_index("c")
    pltpu.sync_copy(offp_hbm, offp_v)
    pltpu.sync_copy(idx_hbm.at[pl.ds(wid * _PER_W, _PER_W)], idxall_v)
    idxp_bufs = (idxp0_v, idxp1_v)
    rows_bufs = (rows0_v, rows1_v)
    g_sems = (sem_g0, sem_g1)
    wb_sems = (sem_w0, sem_w1)
    g = [None] * _NCH
    wb = [None] * _NCH

    def start_gather(c):
        for k in range(_CH // 16):
            sl = pl.ds(k * 16, 16)
            idxp_bufs[c % 2][sl] = (idxall_v[pl.ds(c * _CH + k * 16, 16)]
                                    + offp_v[sl])
        return pltpu.async_copy(table_hbm.at[idxp_bufs[c % 2]],
                                rows_bufs[c % 2], g_sems[c % 2])

    def start_wb(c):
        base = wid * _PER_W + c * _CH
        return pltpu.async_copy(rows_bufs[c % 2],
                                out_hbm.at[pl.ds(base, _CH)], wb_sems[c % 2])

    for c in range(_NCH):
        if c >= 2:
            wb[c - 2].wait()
        g[c] = start_gather(c)
        if c >= 1:
            g[c - 1].wait()
            wb[c - 1] = start_wb(c - 1)
    g[_NCH - 1].wait()
    wb[_NCH - 1] = start_wb(_NCH - 1)
    wb[_NCH - 2].wait()
    wb[_NCH - 1].wait()


def _sc_gather(flat_idx, offp, emb):
    gather = pl.kernel(
        _sc_gather_body,
        out_type=jax.ShapeDtypeStruct((_N_ROWS, D_PER_HEAD), jnp.float32),
        mesh=plsc.VectorSubcoreMesh(core_axis_name="c", subcore_axis_name="s",
                                    num_cores=_NC, num_subcores=_NS),
        scratch_types=[
            pltpu.VMEM((_PER_W,), jnp.int32),
            pltpu.VMEM((_CH,), jnp.int32),
            pltpu.VMEM((_CH,), jnp.int32),
            pltpu.VMEM((_CH,), jnp.int32),
            pltpu.VMEM((_CH, D_PER_HEAD), jnp.float32),
            pltpu.VMEM((_CH, D_PER_HEAD), jnp.float32),
            pltpu.SemaphoreType.DMA,
            pltpu.SemaphoreType.DMA,
            pltpu.SemaphoreType.DMA,
            pltpu.SemaphoreType.DMA,
        ],
    )
    return gather(flat_idx, offp, emb)


_TL = 512
_NT = L // _TL


def _rms(x, g):
    ms = jnp.mean(x * x, axis=-1, keepdims=True)
    return x * lax.rsqrt(ms + EPS) * g


def _gate_vtilde(e, h, wv, wk, ghk):
    dn = (((1,), (1,)), ((), ()))
    kt = lax.dot_general(e, wk, dn, precision=lax.Precision.DEFAULT,
                         preferred_element_type=jnp.float32)
    vt = lax.dot_general(e, wv, dn, precision=lax.Precision.DEFAULT,
                         preferred_element_type=jnp.float32)
    ms_h = jnp.mean(h * h, axis=-1, keepdims=True)
    ms_k = jnp.mean(kt * kt, axis=-1, keepdims=True)
    hk = jnp.sum(h * kt * ghk, axis=-1, keepdims=True)
    dot = hk * lax.rsqrt(ms_h + EPS) * lax.rsqrt(ms_k + EPS) * (D_HIDDEN ** -0.5)
    sdp = jnp.sqrt(jnp.clip(jnp.abs(dot), 1e-6, None)) * jnp.sign(dot)
    return jax.nn.sigmoid(sdp) * vt


def _tc_body(e_ref, ep_ref, h_ref, hp_ref, wv_ref, wk_ref, gh_ref, gk_ref,
             gv_ref, cw_ref, cb_ref, out_ref):
    i = pl.program_id(1)
    wv = wv_ref[...]
    wk = wk_ref[...]
    ghk = gh_ref[...] * gk_ref[...]
    gv = gv_ref[...]

    h = h_ref[0]
    e = jnp.reshape(e_ref[0], (_TL, D_MEM))
    v_tilde = _gate_vtilde(e, h, wv, wk, ghk)

    cw = cw_ref[...]
    cb = cb_ref[...]
    conv_active = jnp.max(jnp.abs(cw)) > 0.0

    @pl.when(jnp.logical_not(conv_active))
    def _skip_conv():
        y = cb * jax.nn.sigmoid(cb)
        out_ref[0] = h + y + v_tilde

    @pl.when(conv_active)
    def _full_conv():
        vn = _rms(v_tilde, gv)
        ep = jnp.reshape(ep_ref[0], (8, D_MEM))
        vtp = _gate_vtilde(ep, hp_ref[0], wv, wk, ghk)
        vnp = _rms(vtp, gv)
        vnp = jnp.where(i == 0, jnp.zeros_like(vnp), vnp)

        ext = jnp.concatenate([vnp, vn], axis=0)
        y = cb
        for j in range(KERNEL):
            y = y + cw[j:j + 1] * ext[5 + j:5 + j + _TL]
        y = y * jax.nn.sigmoid(y)
        out_ref[0] = h + y + v_tilde


def _tc_dense(e3, hidden, wv, wk, gh, gk, gv, cw, cb):
    grid = (B, _NT)
    return pl.pallas_call(
        _tc_body,
        grid=grid,
        in_specs=[
            pl.BlockSpec((1, _TL * NUM_HEADS, D_PER_HEAD),
                         lambda b, i: (b, i, 0)),
            pl.BlockSpec((1, 8 * NUM_HEADS, D_PER_HEAD),
                         lambda b, i: (b, jnp.maximum(i * (_TL // 8) - 1, 0), 0)),
            pl.BlockSpec((1, _TL, D_HIDDEN), lambda b, i: (b, i, 0)),
            pl.BlockSpec((1, 8, D_HIDDEN),
                         lambda b, i: (b, jnp.maximum(i * (_TL // 8) - 1, 0), 0)),
            pl.BlockSpec((D_HIDDEN, D_MEM), lambda b, i: (0, 0)),
            pl.BlockSpec((D_HIDDEN, D_MEM), lambda b, i: (0, 0)),
            pl.BlockSpec((1, D_HIDDEN), lambda b, i: (0, 0)),
            pl.BlockSpec((1, D_HIDDEN), lambda b, i: (0, 0)),
            pl.BlockSpec((1, D_HIDDEN), lambda b, i: (0, 0)),
            pl.BlockSpec((KERNEL, D_HIDDEN), lambda b, i: (0, 0)),
            pl.BlockSpec((1, D_HIDDEN), lambda b, i: (0, 0)),
        ],
        out_specs=pl.BlockSpec((1, _TL, D_HIDDEN), lambda b, i: (b, i, 0)),
        out_shape=jax.ShapeDtypeStruct((B, L, D_HIDDEN), jnp.float32),
    )(e3, e3, hidden, hidden, wv, wk, gh, gk, gv, cw, cb)


def kernel(engram_hash_indices, hidden_states, emb, w_v, w_k, g_h, g_k, g_v,
           conv_w, conv_b):
    flat_idx = engram_hash_indices.reshape(-1)
    rows = _sc_gather(flat_idx, jnp.asarray(_OFFP), emb)
    e4 = rows.reshape(B, L * NUM_HEADS, D_PER_HEAD)
    out = _tc_dense(
        e4,
        hidden_states,
        w_v,
        w_k,
        g_h.reshape(1, D_HIDDEN),
        g_k.reshape(1, D_HIDDEN),
        g_v.reshape(1, D_HIDDEN),
        jnp.transpose(conv_w[:, 0, :], (1, 0)),
        conv_b.reshape(1, D_HIDDEN),
    )
    return out
